# baseline (speedup 1.0000x reference)
import jax
import jax.numpy as jnp
from jax import lax
from jax.experimental import pallas as pl
from jax.experimental.pallas import tpu as pltpu, tpu_sc as plsc

B = 16384
D = 64
LANES = 16
NUM_CORES = 2
NUM_SUBCORES = 16
NW = NUM_CORES * NUM_SUBCORES
BW = B // NW
GROUPS = BW // LANES
SEGS = D // LANES


def _body(user_idx, item_idx, uemb, iemb, ubias8, ibias8, out,
          idx_u, idx_i, idx_u8, idx_i8, u_rows, i_rows, ub8, ib8, out_v,
          sem_u, sem_i, sem_ub, sem_ib):
    wid = lax.axis_index("s") * NUM_CORES + lax.axis_index("c")
    base = wid * BW

    pltpu.sync_copy(user_idx.at[pl.ds(base, BW)], idx_u)
    pltpu.sync_copy(item_idx.at[pl.ds(base, BW)], idx_i)

    def shift_chunk(c, carry):
        sl = pl.ds(c * LANES, LANES)
        idx_u8[sl] = jax.lax.shift_right_logical(idx_u[sl], 3)
        idx_i8[sl] = jax.lax.shift_right_logical(idx_i[sl], 3)
        return carry

    lax.fori_loop(0, BW // LANES, shift_chunk, 0)

    cub = pltpu.async_copy(ubias8.at[idx_u8], ub8, sem_ub)
    cib = pltpu.async_copy(ibias8.at[idx_i8], ib8, sem_ib)
    cu = pltpu.async_copy(uemb.at[idx_u], u_rows, sem_u)
    ci = pltpu.async_copy(iemb.at[idx_i], i_rows, sem_i)
    cub.wait()
    cib.wait()
    cu.wait()
    ci.wait()

    lanes = lax.iota(jnp.int32, LANES)

    def group(g, carry):
        r0 = g * LANES
        sl = pl.ds(r0, LANES)
        uv = idx_u[sl]
        iv = idx_i[sl]
        acc = (plsc.load_gather(ub8, [r0 + lanes, uv & 7]) +
               plsc.load_gather(ib8, [r0 + lanes, iv & 7]))
        for r2 in range(LANES):
            r = r0 + r2
            p = u_rows[r, pl.ds(0, LANES)] * i_rows[r, pl.ds(0, LANES)]
            for j in range(1, SEGS):
                p = p + (u_rows[r, pl.ds(j * LANES, LANES)] *
                         i_rows[r, pl.ds(j * LANES, LANES)])
            acc = acc + jnp.where(lanes == r2, jnp.sum(p), 0.0)
        out_v[sl] = acc
        return carry

    lax.fori_loop(0, GROUPS, group, 0)
    pltpu.sync_copy(out_v, out.at[pl.ds(base, BW)])


def kernel(user, item, user_embeddings, item_embeddings, user_biases, item_biases):
    f = pl.kernel(
        _body,
        out_type=jax.ShapeDtypeStruct((B,), jnp.float32),
        compiler_params=pltpu.CompilerParams(needs_layout_passes=False,
                                             use_tc_tiling_on_sc=False),
        mesh=plsc.VectorSubcoreMesh(core_axis_name="c", subcore_axis_name="s",
                                    num_cores=NUM_CORES,
                                    num_subcores=NUM_SUBCORES),
        scratch_types=[
            pltpu.VMEM((BW,), jnp.int32),
            pltpu.VMEM((BW,), jnp.int32),
            pltpu.VMEM((BW,), jnp.int32),
            pltpu.VMEM((BW,), jnp.int32),
            pltpu.VMEM((BW, D), jnp.float32),
            pltpu.VMEM((BW, D), jnp.float32),
            pltpu.VMEM((BW, 8), jnp.float32),
            pltpu.VMEM((BW, 8), jnp.float32),
            pltpu.VMEM((BW,), jnp.float32),
            pltpu.SemaphoreType.DMA,
            pltpu.SemaphoreType.DMA,
            pltpu.SemaphoreType.DMA,
            pltpu.SemaphoreType.DMA,
        ],
    )
    return f(user, item, user_embeddings, item_embeddings,
             user_biases.reshape(-1, 8), item_biases.reshape(-1, 8))
